# baseline (speedup 1.0000x reference)
import functools
import math

import jax
import jax.numpy as jnp
import numpy as np
from jax.experimental import pallas as pl
from jax.experimental.pallas import tpu as pltpu


def _denoiser_body(
    t_ref, traj_ref, act_ref,
    fw2_ref, wt1t_ref, wt1sc_ref, bt1_ref, wt2_ref, bt2_ref,
    bp_ref, wpt_ref, wpa_ref, b0_ref, w0_ref,
    lng_ref, lnb_ref, wr_ref, br_ref,
    lnfg_ref, lnfb_ref, wf_ref, bfin_ref,
    out_ref,
    *, num_layers: int, eps: float,
):
    f32 = jnp.float32
    n_sub = 2
    sub = t_ref.shape[0] // n_sub

    def ln_relu(v, g, b):
        mu = jnp.mean(v, axis=-1, keepdims=True)
        var = jnp.mean(jnp.square(v - mu), axis=-1, keepdims=True)
        return jnp.maximum((v - mu) * jax.lax.rsqrt(var + eps) * g + b, 0.0)

    def mm(a, w_ref):
        return jnp.dot(a, w_ref[...], preferred_element_type=f32)

    for s in range(n_sub):
        rows = pl.ds(s * sub, sub)
        t = t_ref[rows, :]
        sc = jnp.sin(t * fw2_ref[0] + fw2_ref[1])
        h1 = t * wt1t_ref[...] + mm(sc, wt1sc_ref) + bt1_ref[...]
        h1 = h1 * (1.0 / (1.0 + jnp.exp(-h1)))
        te = mm(h1, wt2_ref) + bt2_ref[...]

        z = (mm(traj_ref[rows, :], wpt_ref) + mm(act_ref[rows, :], wpa_ref)
             + bp_ref[...] + te)

        h = mm(z, w0_ref) + b0_ref[...]
        for i in range(num_layers):
            a = ln_relu(h, lng_ref[i], lnb_ref[i])
            h = h + mm(a, wr_ref[i]) + br_ref[i]
        a = ln_relu(h, lnfg_ref[...], lnfb_ref[...])
        out_ref[rows, :] = (mm(a, wf_ref) + bfin_ref[...]).astype(out_ref.dtype)


def kernel(traj, act, timesteps, fourier_w, wt1, bt1, wt2, bt2, wp, bp,
           w0, b0, ln_g, ln_b, wr, br, lnf_g, lnf_b, wf, bf):
    f32 = jnp.float32
    b, hor, d = traj.shape
    dc = act.shape[-1]
    trajf = traj.reshape(b, hor * d).astype(f32)
    actf = act.reshape(b, hor * dc).astype(f32)
    t = timesteps.reshape(b, 1).astype(f32)

    E = wt2.shape[0]
    H = w0.shape[1]
    L = wr.shape[0]
    dout = wf.shape[1]
    half = fourier_w.shape[0]

    rows = np.arange(hor * (d + dc)).reshape(hor, d + dc)
    wp_t = wp[np.asarray(rows[:, :d].reshape(-1))]
    wp_a = wp[np.asarray(rows[:, d:].reshape(-1))]

    tb = 1024 if b >= 1024 else max(8, ((b + 7) // 8) * 8)
    b_pad = ((b + tb - 1) // tb) * tb
    if b_pad != b:
        trajf = jnp.pad(trajf, ((0, b_pad - b), (0, 0)))
        actf = jnp.pad(actf, ((0, b_pad - b), (0, 0)))
        t = jnp.pad(t, ((0, b_pad - b), (0, 0)))

    fw_rep = jnp.tile(fourier_w.reshape(1, half) * (2.0 * math.pi), (1, 2))
    offs = jnp.concatenate(
        [jnp.zeros((1, half), f32), jnp.full((1, half), 0.5 * math.pi, f32)],
        axis=1)
    fw2 = jnp.concatenate([fw_rep, offs], axis=0)

    def row(v):
        return v.reshape(1, -1)

    weight_inputs = [
        fw2,
        wt1[0:1, :], wt1[1:, :], row(bt1),
        wt2, row(bt2),
        row(bp), wp_t, wp_a,
        row(b0), w0,
        ln_g.reshape(L, 1, H), ln_b.reshape(L, 1, H),
        wr, br.reshape(L, 1, H),
        row(lnf_g), row(lnf_b),
        wf, row(bf),
    ]

    def const_spec(a):
        return pl.BlockSpec(a.shape, lambda i: (0,) * a.ndim)

    in_specs = (
        [pl.BlockSpec((tb, 1), lambda i: (i, 0)),
         pl.BlockSpec((tb, hor * d), lambda i: (i, 0)),
         pl.BlockSpec((tb, hor * dc), lambda i: (i, 0))]
        + [const_spec(a) for a in weight_inputs]
    )

    body = functools.partial(_denoiser_body, num_layers=L, eps=1e-5)
    y = pl.pallas_call(
        body,
        out_shape=jax.ShapeDtypeStruct((b_pad, dout), f32),
        grid=(b_pad // tb,),
        in_specs=in_specs,
        out_specs=pl.BlockSpec((tb, dout), lambda i: (i, 0)),
        compiler_params=pltpu.CompilerParams(
            dimension_semantics=("parallel",),
        ),
    )(t, trajf, actf, *weight_inputs)
    return y[:b].reshape(b, hor, d)


# speedup vs baseline: 1.0476x; 1.0476x over previous
import functools
import math

import jax
import jax.numpy as jnp
import numpy as np
from jax.experimental import pallas as pl
from jax.experimental.pallas import tpu as pltpu


def _denoiser_body(
    t_ref, traj_ref, act_ref,
    fw2_ref, wt1t_ref, wt1sc_ref, bt1_ref, wt2_ref, bt2_ref,
    bp_ref, wpt_ref, wpa_ref, b0_ref, w0_ref,
    lng_ref, lnb_ref, wr_hbm, br_ref,
    lnfg_ref, lnfb_ref, wf_hbm, bfin_ref,
    out_ref,
    wr_scr, wf_scr, sems,
    *, num_layers: int, eps: float,
):
    f32 = jnp.float32

    for i in range(num_layers):
        pltpu.make_async_copy(wr_hbm.at[i], wr_scr.at[i], sems.at[i]).start()
    pltpu.make_async_copy(wf_hbm, wf_scr, sems.at[num_layers]).start()

    def ln_relu(v, g, b):
        mu = jnp.mean(v, axis=-1, keepdims=True)
        var = jnp.mean(jnp.square(v - mu), axis=-1, keepdims=True)
        return jnp.maximum((v - mu) * jax.lax.rsqrt(var + eps) * g + b, 0.0)

    def mm(a, w_ref):
        return jnp.dot(a, w_ref[...], preferred_element_type=f32)

    t = t_ref[...]
    sc = jnp.sin(t * fw2_ref[0] + fw2_ref[1])
    h1 = t * wt1t_ref[...] + mm(sc, wt1sc_ref) + bt1_ref[...]
    h1 = h1 * (1.0 / (1.0 + jnp.exp(-h1)))
    te = mm(h1, wt2_ref) + bt2_ref[...]

    z = (mm(traj_ref[...], wpt_ref) + mm(act_ref[...], wpa_ref)
         + bp_ref[...] + te)

    h = mm(z, w0_ref) + b0_ref[...]
    for i in range(num_layers):
        a = ln_relu(h, lng_ref[i], lnb_ref[i])
        pltpu.make_async_copy(wr_scr.at[i], wr_scr.at[i], sems.at[i]).wait()
        h = h + mm(a, wr_scr.at[i]) + br_ref[i]
    a = ln_relu(h, lnfg_ref[...], lnfb_ref[...])
    pltpu.make_async_copy(wf_scr, wf_scr, sems.at[num_layers]).wait()
    out_ref[...] = (mm(a, wf_scr) + bfin_ref[...]).astype(out_ref.dtype)


def kernel(traj, act, timesteps, fourier_w, wt1, bt1, wt2, bt2, wp, bp,
           w0, b0, ln_g, ln_b, wr, br, lnf_g, lnf_b, wf, bf):
    f32 = jnp.float32
    b, hor, d = traj.shape
    dc = act.shape[-1]
    trajf = traj.reshape(b, hor * d).astype(f32)
    actf = act.reshape(b, hor * dc).astype(f32)
    t = timesteps.reshape(b, 1).astype(f32)

    E = wt2.shape[0]
    H = w0.shape[1]
    L = wr.shape[0]
    dout = wf.shape[1]
    half = fourier_w.shape[0]

    rows = np.arange(hor * (d + dc)).reshape(hor, d + dc)
    wp_t = wp[np.asarray(rows[:, :d].reshape(-1))]
    wp_a = wp[np.asarray(rows[:, d:].reshape(-1))]

    tb = 1024 if b >= 1024 else max(8, ((b + 7) // 8) * 8)
    b_pad = ((b + tb - 1) // tb) * tb
    if b_pad != b:
        trajf = jnp.pad(trajf, ((0, b_pad - b), (0, 0)))
        actf = jnp.pad(actf, ((0, b_pad - b), (0, 0)))
        t = jnp.pad(t, ((0, b_pad - b), (0, 0)))

    fw_rep = jnp.tile(fourier_w.reshape(1, half) * (2.0 * math.pi), (1, 2))
    offs = jnp.concatenate(
        [jnp.zeros((1, half), f32), jnp.full((1, half), 0.5 * math.pi, f32)],
        axis=1)
    fw2 = jnp.concatenate([fw_rep, offs], axis=0)

    def row(v):
        return v.reshape(1, -1)

    weight_inputs = [
        fw2,
        wt1[0:1, :], wt1[1:, :], row(bt1),
        wt2, row(bt2),
        row(bp), wp_t, wp_a,
        row(b0), w0,
        ln_g.reshape(L, 1, H), ln_b.reshape(L, 1, H),
        wr, br.reshape(L, 1, H),
        row(lnf_g), row(lnf_b),
        wf, row(bf),
    ]

    def const_spec(a):
        return pl.BlockSpec(a.shape, lambda i: (0,) * a.ndim)

    weight_specs = [const_spec(a) for a in weight_inputs]
    weight_specs[13] = pl.BlockSpec(memory_space=pl.ANY)
    weight_specs[17] = pl.BlockSpec(memory_space=pl.ANY)

    in_specs = (
        [pl.BlockSpec((tb, 1), lambda i: (i, 0)),
         pl.BlockSpec((tb, hor * d), lambda i: (i, 0)),
         pl.BlockSpec((tb, hor * dc), lambda i: (i, 0))]
        + weight_specs
    )

    body = functools.partial(_denoiser_body, num_layers=L, eps=1e-5)
    y = pl.pallas_call(
        body,
        out_shape=jax.ShapeDtypeStruct((b_pad, dout), f32),
        grid=(b_pad // tb,),
        in_specs=in_specs,
        out_specs=pl.BlockSpec((tb, dout), lambda i: (i, 0)),
        scratch_shapes=[
            pltpu.VMEM((L, H, H), f32),
            pltpu.VMEM((H, dout), f32),
            pltpu.SemaphoreType.DMA((L + 1,)),
        ],
        compiler_params=pltpu.CompilerParams(
            dimension_semantics=("parallel",),
        ),
    )(t, trajf, actf, *weight_inputs)
    return y[:b].reshape(b, hor, d)
